# baseline (speedup 1.0000x reference)
import functools

import jax
import jax.numpy as jnp
from jax import lax
from jax.experimental import pallas as pl
from jax.experimental.pallas import tpu as pltpu
from jax.experimental.pallas import tpu_sc as plsc

_NC = 2
_NS = 16
_NW = _NC * _NS


def _sc_pool(inputT, W, batch, bag, dim):
    bpw = batch // _NW
    mesh = plsc.VectorSubcoreMesh(core_axis_name="c", subcore_axis_name="s")

    @functools.partial(
        pl.kernel,
        out_type=jax.ShapeDtypeStruct((batch, dim), jnp.float32),
        mesh=mesh,
        scratch_types=[
            pltpu.VMEM((bpw,), jnp.int32),
            pltpu.VMEM((bpw, dim), jnp.float32),
            pltpu.SemaphoreType.DMA,
        ],
        compiler_params=pltpu.CompilerParams(use_tc_tiling_on_sc=False),
    )
    def body(inputT_hbm, w_hbm, out_hbm, idx_v, acc_v, sem):
        wid = lax.axis_index("s") * _NC + lax.axis_index("c")
        base = wid * bpw

        pltpu.sync_copy(inputT_hbm.at[0, pl.ds(base, bpw)], idx_v)
        pltpu.async_copy(w_hbm.at[idx_v], acc_v, sem).wait()

        @pl.loop(1, bag)
        def _(j):
            pltpu.sync_copy(inputT_hbm.at[j, pl.ds(base, bpw)], idx_v)
            pltpu.async_copy(w_hbm.at[idx_v], acc_v, sem, add=True).wait()

        pltpu.sync_copy(acc_v, out_hbm.at[pl.ds(base, bpw)])

    return body(inputT, W)


def _tc_proj(pooled, P, batch, dim, out_dim):
    blk = 1024

    def body(x_ref, p_ref, o_ref):
        o_ref[...] = lax.dot_general(
            x_ref[...], p_ref[...],
            (((1,), (1,)), ((), ())),
            preferred_element_type=jnp.float32,
        )

    return pl.pallas_call(
        body,
        grid=(batch // blk,),
        in_specs=[
            pl.BlockSpec((blk, dim), lambda i: (i, 0)),
            pl.BlockSpec((out_dim, dim), lambda i: (0, 0)),
        ],
        out_specs=pl.BlockSpec((blk, out_dim), lambda i: (i, 0)),
        out_shape=jax.ShapeDtypeStruct((batch, out_dim), jnp.float32),
    )(pooled, P)


def kernel(input, W, P):
    batch, bag = input.shape
    _, dim = W.shape
    out_dim = P.shape[0]
    inputT = input.astype(jnp.int32).T
    pooled = _sc_pool(inputT, W, batch, bag, dim)
    return _tc_proj(pooled, P, batch, dim, out_dim)


# speedup vs baseline: 1.0991x; 1.0991x over previous
import functools

import jax
import jax.numpy as jnp
from jax import lax
from jax.experimental import pallas as pl
from jax.experimental.pallas import tpu as pltpu
from jax.experimental.pallas import tpu_sc as plsc

_NC = 2
_NS = 16
_NW = _NC * _NS


def _sc_pool(inputT, W, batch, bag, dim):
    bpw = batch // _NW
    mesh = plsc.VectorSubcoreMesh(core_axis_name="c", subcore_axis_name="s")

    @functools.partial(
        pl.kernel,
        out_type=jax.ShapeDtypeStruct((batch, dim), jnp.float32),
        mesh=mesh,
        scratch_types=[
            pltpu.VMEM((bag, bpw), jnp.int32),
            pltpu.VMEM((bpw, dim), jnp.float32),
            pltpu.SemaphoreType.DMA,
        ],
        compiler_params=pltpu.CompilerParams(use_tc_tiling_on_sc=False),
    )
    def body(inputT_hbm, w_hbm, out_hbm, idx_v, acc_v, sem):
        wid = lax.axis_index("s") * _NC + lax.axis_index("c")
        base = wid * bpw

        pltpu.sync_copy(inputT_hbm.at[:, pl.ds(base, bpw)], idx_v)

        zeros = jnp.zeros((16,), jnp.float32)

        @pl.loop(0, bpw)
        def _(i):
            for h in range(dim // 16):
                acc_v[i, pl.ds(h * 16, 16)] = zeros

        copies = [
            pltpu.async_copy(w_hbm.at[idx_v.at[j]], acc_v, sem, add=True)
            for j in range(bag)
        ]
        for c in copies:
            c.wait()

        pltpu.sync_copy(acc_v, out_hbm.at[pl.ds(base, bpw)])

    return body(inputT, W)


def _tc_proj(pooled, P, batch, dim, out_dim):
    blk = 1024

    def body(x_ref, p_ref, o_ref):
        o_ref[...] = lax.dot_general(
            x_ref[...], p_ref[...],
            (((1,), (1,)), ((), ())),
            preferred_element_type=jnp.float32,
        )

    return pl.pallas_call(
        body,
        grid=(batch // blk,),
        in_specs=[
            pl.BlockSpec((blk, dim), lambda i: (i, 0)),
            pl.BlockSpec((out_dim, dim), lambda i: (0, 0)),
        ],
        out_specs=pl.BlockSpec((blk, out_dim), lambda i: (i, 0)),
        out_shape=jax.ShapeDtypeStruct((batch, out_dim), jnp.float32),
    )(pooled, P)


def kernel(input, W, P):
    batch, bag = input.shape
    _, dim = W.shape
    out_dim = P.shape[0]
    inputT = input.astype(jnp.int32).T
    pooled = _sc_pool(inputT, W, batch, bag, dim)
    return _tc_proj(pooled, P, batch, dim, out_dim)
